# baseline (speedup 1.0000x reference)
import functools

import jax
import jax.numpy as jnp
from jax import lax
from jax.experimental import pallas as pl
from jax.experimental.pallas import tpu as pltpu
from jax.experimental.pallas import tpu_sc as plsc

EMBED_DIM = 32
LANES = 128
NUM_CORES = 2
NUM_SUBCORES = 16
NUM_WORKERS = NUM_CORES * NUM_SUBCORES
NBUF = 8
ROWS_PER_CHUNK = LANES * EMBED_DIM // LANES


def _make_gather(n_idx: int):
  idx_rows = n_idx // LANES
  rows_per_w = idx_rows // NUM_WORKERS
  n_groups = rows_per_w // NBUF
  out_rows = n_idx * EMBED_DIM // LANES
  mesh = plsc.VectorSubcoreMesh(core_axis_name="c", subcore_axis_name="s")

  @functools.partial(
      pl.kernel,
      mesh=mesh,
      compiler_params=pltpu.CompilerParams(use_tc_tiling_on_sc=False),
      out_type=jax.ShapeDtypeStruct((out_rows, LANES), jnp.float32),
      scratch_types=(
          [pltpu.VMEM((rows_per_w, LANES), jnp.int32)]
          + [pltpu.VMEM((LANES, EMBED_DIM), jnp.float32) for _ in range(NBUF)]
          + [pltpu.VMEM((ROWS_PER_CHUNK, LANES), jnp.float32)
             for _ in range(NBUF)]
          + [pltpu.SemaphoreType.DMA for _ in range(2 * NBUF)]
      ),
  )
  def k(table_hbm, idx_hbm, out_hbm, idx_v, *bufs_and_sems):
    bufa = bufs_and_sems[:NBUF]
    bufb = bufs_and_sems[NBUF:2 * NBUF]
    gsem = bufs_and_sems[2 * NBUF:3 * NBUF]
    ssem = bufs_and_sems[3 * NBUF:]
    wid = lax.axis_index("s") * NUM_CORES + lax.axis_index("c")
    chunk_base = wid * rows_per_w

    pltpu.sync_copy(idx_hbm.at[pl.ds(chunk_base, rows_per_w), :], idx_v)

    def start_gather(c, b):
      pltpu.async_copy(table_hbm.at[idx_v.at[c]], bufa[b], gsem[b])

    def wait_gather(b):
      pltpu.make_async_copy(
          table_hbm.at[idx_v.at[0]], bufa[b], gsem[b]).wait()

    def repack(b):
      for r in range(ROWS_PER_CHUNK):
        for kk in range(LANES // 16):
          bufb[b][r, pl.ds(16 * kk, 16)] = (
              bufa[b][4 * r + kk // 2, pl.ds(16 * (kk % 2), 16)])

    def start_store(c, b):
      pltpu.async_copy(
          bufb[b],
          out_hbm.at[pl.ds((chunk_base + c) * ROWS_PER_CHUNK, ROWS_PER_CHUNK),
                     :],
          ssem[b])

    def wait_store(b):
      pltpu.make_async_copy(
          bufb[b],
          out_hbm.at[pl.ds(0, ROWS_PER_CHUNK), :], ssem[b]).wait()

    for b in range(NBUF):
      start_gather(b, b)

    def body(j, carry):
      c0 = j * NBUF
      for b in range(NBUF):
        wait_gather(b)
        repack(b)
        start_store(c0 + b, b)
      for b in range(NBUF):
        wait_store(b)
        start_gather(c0 + NBUF + b, b)
      return carry

    lax.fori_loop(0, n_groups - 1, body, 0)

    c0 = (n_groups - 1) * NBUF
    for b in range(NBUF):
      wait_gather(b)
      repack(b)
      start_store(c0 + b, b)
    for b in range(NBUF):
      wait_store(b)

  return k


def kernel(indices, table):
  b, h = indices.shape
  idx128 = indices.reshape(b * h // LANES, LANES)
  gather = _make_gather(b * h)
  out = gather(table, idx128)
  return out.reshape(b, h, EMBED_DIM)
